# Initial kernel scaffold; baseline (speedup 1.0000x reference)
import jax
import jax.numpy as jnp
from jax.experimental import pallas as pl


def kernel(x, gate_w, gate_b, expert_w, expert_b):
    raise NotImplementedError("write your pallas kernel here")


# speedup vs baseline: 3.6701x; 3.6701x over previous
import jax
import jax.numpy as jnp
from jax import lax
from jax.experimental import pallas as pl
from jax.experimental.pallas import tpu as pltpu
from jax.experimental.pallas import tpu_sc as plsc

NE = 64
NTOK = 2048
D = 768
BT = 128
NB = NTOK // BT
S = NB + NE
NC = 2
NS = 16
NW = NC * NS
TPW = NTOK // NW
L = 16
NTP = (NB + 1) * BT
BIG = 2**30


def _gate_body(x_ref, gw_ref, gb_ref, pos_ref, prob_ref, counts_ref):
    x = x_ref[...]
    scores = lax.dot_general(x, gw_ref[...], (((1,), (1,)), ((), ())),
                             preferred_element_type=jnp.float32)
    scores = scores + gb_ref[...]
    m = jnp.max(scores, axis=1, keepdims=True)
    s = jnp.sum(jnp.exp(scores - m), axis=1, keepdims=True)
    prob_ref[...] = 1.0 / s
    lane = lax.broadcasted_iota(jnp.int32, scores.shape, 1)
    eid = jnp.min(jnp.where(scores == m, lane, NE), axis=1, keepdims=True)
    oh = (lane == eid).astype(jnp.float32)
    counts = jnp.sum(oh, axis=0, keepdims=True)
    counts_ref[...] = counts.astype(jnp.int32)
    ei = lax.broadcasted_iota(jnp.int32, (NE, NE), 0)
    ej = lax.broadcasted_iota(jnp.int32, (NE, NE), 1)
    su = (ei < ej).astype(jnp.float32)
    starts = lax.dot_general(counts, su, (((1,), (0,)), ((), ())),
                             preferred_element_type=jnp.float32)
    ri = lax.broadcasted_iota(jnp.int32, (BT, BT), 0)
    rj = lax.broadcasted_iota(jnp.int32, (BT, BT), 1)
    tl = (rj < ri).astype(jnp.float32)
    run = jnp.zeros((1, NE), jnp.float32)
    for b in range(NB):
        blk = oh[b * BT:(b + 1) * BT]
        rank = lax.dot_general(tl, blk, (((1,), (0,)), ((), ())),
                               preferred_element_type=jnp.float32) + run
        pos = jnp.sum(blk * (starts + rank), axis=1, keepdims=True)
        pos_ref[b * BT:(b + 1) * BT, :] = pos.astype(jnp.int32)
        run = run + jnp.sum(blk, axis=0, keepdims=True)


def _gate(x, gate_w, gate_b):
    return pl.pallas_call(
        _gate_body,
        out_shape=[
            jax.ShapeDtypeStruct((NTOK, 1), jnp.int32),
            jax.ShapeDtypeStruct((NTOK, 1), jnp.float32),
            jax.ShapeDtypeStruct((1, NE), jnp.int32),
        ],
    )(x, gate_w, gate_b.reshape(1, NE))


def _sc_mesh():
    return plsc.VectorSubcoreMesh(core_axis_name="c", subcore_axis_name="s",
                                  num_cores=NC, num_subcores=NS)


def _dispatch_body(x_hbm, pos_hbm, prob_hbm, xs_hbm, ps_hbm,
                   pos_v, pstage, xrows_v, sem_a, sem_b):
    wid = lax.axis_index("s") * NC + lax.axis_index("c")
    base = wid * TPW
    pltpu.sync_copy(x_hbm.at[pl.ds(base, TPW)], xrows_v)
    pltpu.sync_copy(pos_hbm.at[pl.ds(base, TPW)], pos_v)
    pltpu.sync_copy(prob_hbm.at[pl.ds(base, TPW)], pstage)
    cps = []
    for c in range(TPW // L):
        destv = pos_v[pl.ds(c * L, L)]
        cps.append(pltpu.async_copy(xrows_v.at[pl.ds(c * L, L)],
                                    xs_hbm.at[destv], sem_a))
        cps.append(pltpu.async_copy(pstage.at[pl.ds(c * L, L)],
                                    ps_hbm.at[destv], sem_b))
    for cp in cps:
        cp.wait()


def _dispatch(x, pos, prob):
    f = pl.kernel(
        _dispatch_body,
        out_type=(
            jax.ShapeDtypeStruct((NTP, D), jnp.float32),
            jax.ShapeDtypeStruct((NTP,), jnp.float32),
        ),
        mesh=_sc_mesh(),
        scratch_types=[
            pltpu.VMEM((TPW,), jnp.int32),
            pltpu.VMEM((TPW,), jnp.float32),
            pltpu.VMEM((TPW, D), jnp.float32),
            pltpu.SemaphoreType.DMA,
            pltpu.SemaphoreType.DMA,
        ],
    )
    return f(x, pos, prob)


def _mm_body(meta_ref, xs_ref, w_ref, b_ref, ps_ref, y_ref):
    s = pl.program_id(0)
    b = meta_ref[s, 0]
    lo = meta_ref[s, 2]
    hi = meta_ref[s, 3]
    rows = b * BT + lax.broadcasted_iota(jnp.int32, (BT, 1), 0)
    mask = (rows >= lo) & (rows < hi)
    y = lax.dot_general(xs_ref[...], w_ref[0], (((1,), (1,)), ((), ())),
                        preferred_element_type=jnp.float32)
    y = (y + b_ref[0]) * ps_ref[...]
    y_ref[...] = jnp.where(mask, y, y_ref[...])


def _grouped_mm(xs, ps, expert_w, expert_b, meta):
    grid_spec = pltpu.PrefetchScalarGridSpec(
        num_scalar_prefetch=1,
        grid=(S,),
        in_specs=[
            pl.BlockSpec((BT, D), lambda s, meta: (meta[s, 0], 0)),
            pl.BlockSpec((1, D, D), lambda s, meta: (meta[s, 1], 0, 0)),
            pl.BlockSpec((1, 1, D), lambda s, meta: (meta[s, 1], 0, 0)),
            pl.BlockSpec((BT, 1), lambda s, meta: (meta[s, 0], 0)),
        ],
        out_specs=pl.BlockSpec((BT, D), lambda s, meta: (meta[s, 0], 0)),
    )
    return pl.pallas_call(
        _mm_body,
        grid_spec=grid_spec,
        out_shape=jax.ShapeDtypeStruct((NTP, D), jnp.float32),
    )(meta, xs, expert_w, expert_b.reshape(NE, 1, D), ps.reshape(NTP, 1))


def _combine_body(y_hbm, pos_hbm, out_hbm, pos_v, rows_v, sem_a):
    wid = lax.axis_index("s") * NC + lax.axis_index("c")
    base = wid * TPW
    pltpu.sync_copy(pos_hbm.at[pl.ds(base, TPW)], pos_v)
    cps = []
    for c in range(TPW // L):
        srcv = pos_v[pl.ds(c * L, L)]
        cps.append(pltpu.async_copy(y_hbm.at[srcv],
                                    rows_v.at[pl.ds(c * L, L)], sem_a))
    for cp in cps:
        cp.wait()
    pltpu.sync_copy(rows_v, out_hbm.at[pl.ds(base, TPW)])


def _combine(y, pos):
    f = pl.kernel(
        _combine_body,
        out_type=jax.ShapeDtypeStruct((NTOK, D), jnp.float32),
        mesh=_sc_mesh(),
        scratch_types=[
            pltpu.VMEM((TPW,), jnp.int32),
            pltpu.VMEM((TPW, D), jnp.float32),
            pltpu.SemaphoreType.DMA,
        ],
    )
    return f(y, pos)


def _make_meta(counts, starts, ends):
    nonempty = counts > 0
    lob = starts // BT
    hib = (ends - 1) // BT
    p = jnp.where(nonempty, hib - lob + 1, 0)
    q = jnp.cumsum(p) - p
    r = jnp.where(nonempty, q, BIG)
    svec = jnp.arange(S, dtype=jnp.int32)
    es = jnp.sum((r[None, :] <= svec[:, None]).astype(jnp.int32), axis=1) - 1
    j = svec - q[es]
    valid_s = j < p[es]
    bs = jnp.where(valid_s, lob[es] + j, NB - 1)
    los = jnp.where(valid_s, jnp.maximum(starts[es], bs * BT), 0)
    his = jnp.where(valid_s, jnp.minimum(ends[es], (bs + 1) * BT), 0)
    return jnp.stack([bs, es, los, his], axis=1).astype(jnp.int32)


def kernel(x, gate_w, gate_b, expert_w, expert_b):
    pos2, prob2, counts2 = _gate(x, gate_w, gate_b)
    pos = pos2[:, 0]
    prob = prob2[:, 0]
    counts = counts2[0]

    ends = jnp.cumsum(counts)
    starts = ends - counts

    xs, ps = _dispatch(x, pos, prob)
    meta = _make_meta(counts, starts, ends)
    y = _grouped_mm(xs, ps, expert_w, expert_b, meta)
    return _combine(y, pos)


# speedup vs baseline: 3.6797x; 1.0026x over previous
import jax
import jax.numpy as jnp
from jax import lax
from jax.experimental import pallas as pl
from jax.experimental.pallas import tpu as pltpu
from jax.experimental.pallas import tpu_sc as plsc

NE = 64
NTOK = 2048
D = 768
BT = 128
NB = NTOK // BT
S = NB + NE
NC = 2
NS = 16
NW = NC * NS
TPW = NTOK // NW
L = 16
NTP = (NB + 1) * BT
BIG = 2**30


def _gate_body(x_ref, gw_ref, gb_ref, pos_ref, prob_ref, counts_ref):
    x = x_ref[...]
    scores = lax.dot_general(x, gw_ref[...], (((1,), (1,)), ((), ())),
                             preferred_element_type=jnp.float32)
    scores = scores + gb_ref[...]
    m = jnp.max(scores, axis=1, keepdims=True)
    s = jnp.sum(jnp.exp(scores - m), axis=1, keepdims=True)
    prob_ref[...] = 1.0 / s
    lane = lax.broadcasted_iota(jnp.int32, scores.shape, 1)
    eid = jnp.min(jnp.where(scores == m, lane, NE), axis=1, keepdims=True)
    oh = (lane == eid).astype(jnp.float32)
    counts = jnp.sum(oh, axis=0, keepdims=True)
    counts_ref[...] = counts.astype(jnp.int32)
    ei = lax.broadcasted_iota(jnp.int32, (NE, NE), 0)
    ej = lax.broadcasted_iota(jnp.int32, (NE, NE), 1)
    su = (ei < ej).astype(jnp.float32)
    starts = lax.dot_general(counts, su, (((1,), (0,)), ((), ())),
                             preferred_element_type=jnp.float32)
    ri = lax.broadcasted_iota(jnp.int32, (BT, BT), 0)
    rj = lax.broadcasted_iota(jnp.int32, (BT, BT), 1)
    tl = (rj < ri).astype(jnp.float32)
    run = jnp.zeros((1, NE), jnp.float32)
    for b in range(NB):
        blk = oh[b * BT:(b + 1) * BT]
        rank = lax.dot_general(tl, blk, (((1,), (0,)), ((), ())),
                               preferred_element_type=jnp.float32) + run
        pos = jnp.sum(blk * (starts + rank), axis=1, keepdims=True)
        pos_ref[b * BT:(b + 1) * BT, :] = pos.astype(jnp.int32)
        run = run + jnp.sum(blk, axis=0, keepdims=True)


def _gate(x, gate_w, gate_b):
    return pl.pallas_call(
        _gate_body,
        out_shape=[
            jax.ShapeDtypeStruct((NTOK, 1), jnp.int32),
            jax.ShapeDtypeStruct((NTOK, 1), jnp.float32),
            jax.ShapeDtypeStruct((1, NE), jnp.int32),
        ],
    )(x, gate_w, gate_b.reshape(1, NE))


def _sc_mesh():
    return plsc.VectorSubcoreMesh(core_axis_name="c", subcore_axis_name="s",
                                  num_cores=NC, num_subcores=NS)


def _dispatch_body(x_hbm, pos_hbm, prob_hbm, xs_hbm, ps_hbm,
                   pos_v, pstage, xrows_v, sem_a, sem_b):
    wid = lax.axis_index("s") * NC + lax.axis_index("c")
    base = wid * TPW
    c1 = pltpu.async_copy(x_hbm.at[pl.ds(base, TPW)], xrows_v, sem_a)
    c2 = pltpu.async_copy(pos_hbm.at[pl.ds(base, TPW)], pos_v, sem_b)
    c3 = pltpu.async_copy(prob_hbm.at[pl.ds(base, TPW)], pstage, sem_b)
    c1.wait()
    c2.wait()
    c3.wait()
    c4 = pltpu.async_copy(xrows_v, xs_hbm.at[pos_v], sem_a)
    c5 = pltpu.async_copy(pstage, ps_hbm.at[pos_v], sem_b)
    c4.wait()
    c5.wait()


def _dispatch(x, pos, prob):
    f = pl.kernel(
        _dispatch_body,
        out_type=(
            jax.ShapeDtypeStruct((NTP, D), jnp.float32),
            jax.ShapeDtypeStruct((NTP,), jnp.float32),
        ),
        mesh=_sc_mesh(),
        scratch_types=[
            pltpu.VMEM((TPW,), jnp.int32),
            pltpu.VMEM((TPW,), jnp.float32),
            pltpu.VMEM((TPW, D), jnp.float32),
            pltpu.SemaphoreType.DMA,
            pltpu.SemaphoreType.DMA,
        ],
    )
    return f(x, pos, prob)


def _mm_body(meta_ref, xs_ref, w_ref, b_ref, ps_ref, y_ref):
    s = pl.program_id(0)
    b = meta_ref[s, 0]
    lo = meta_ref[s, 2]
    hi = meta_ref[s, 3]
    rows = b * BT + lax.broadcasted_iota(jnp.int32, (BT, 1), 0)
    mask = (rows >= lo) & (rows < hi)
    y = lax.dot_general(xs_ref[...], w_ref[0], (((1,), (1,)), ((), ())),
                        preferred_element_type=jnp.float32)
    y = (y + b_ref[0]) * ps_ref[...]
    y_ref[...] = jnp.where(mask, y, y_ref[...])


def _grouped_mm(xs, ps, expert_w, expert_b, meta):
    grid_spec = pltpu.PrefetchScalarGridSpec(
        num_scalar_prefetch=1,
        grid=(S,),
        in_specs=[
            pl.BlockSpec((BT, D), lambda s, meta: (meta[s, 0], 0)),
            pl.BlockSpec((1, D, D), lambda s, meta: (meta[s, 1], 0, 0)),
            pl.BlockSpec((1, 1, D), lambda s, meta: (meta[s, 1], 0, 0)),
            pl.BlockSpec((BT, 1), lambda s, meta: (meta[s, 0], 0)),
        ],
        out_specs=pl.BlockSpec((BT, D), lambda s, meta: (meta[s, 0], 0)),
    )
    return pl.pallas_call(
        _mm_body,
        grid_spec=grid_spec,
        out_shape=jax.ShapeDtypeStruct((NTP, D), jnp.float32),
    )(meta, xs, expert_w, expert_b.reshape(NE, 1, D), ps.reshape(NTP, 1))


def _combine_body(y_hbm, pos_hbm, out_hbm, pos_v, rows_v, sem_a):
    wid = lax.axis_index("s") * NC + lax.axis_index("c")
    base = wid * TPW
    pltpu.sync_copy(pos_hbm.at[pl.ds(base, TPW)], pos_v)
    pltpu.async_copy(y_hbm.at[pos_v], rows_v, sem_a).wait()
    pltpu.sync_copy(rows_v, out_hbm.at[pl.ds(base, TPW)])


def _combine(y, pos):
    f = pl.kernel(
        _combine_body,
        out_type=jax.ShapeDtypeStruct((NTOK, D), jnp.float32),
        mesh=_sc_mesh(),
        scratch_types=[
            pltpu.VMEM((TPW,), jnp.int32),
            pltpu.VMEM((TPW, D), jnp.float32),
            pltpu.SemaphoreType.DMA,
        ],
    )
    return f(y, pos)


def _make_meta(counts, starts, ends):
    nonempty = counts > 0
    lob = starts // BT
    hib = (ends - 1) // BT
    p = jnp.where(nonempty, hib - lob + 1, 0)
    q = jnp.cumsum(p) - p
    r = jnp.where(nonempty, q, BIG)
    svec = jnp.arange(S, dtype=jnp.int32)
    es = jnp.sum((r[None, :] <= svec[:, None]).astype(jnp.int32), axis=1) - 1
    j = svec - q[es]
    valid_s = j < p[es]
    bs = jnp.where(valid_s, lob[es] + j, NB - 1)
    los = jnp.where(valid_s, jnp.maximum(starts[es], bs * BT), 0)
    his = jnp.where(valid_s, jnp.minimum(ends[es], (bs + 1) * BT), 0)
    return jnp.stack([bs, es, los, his], axis=1).astype(jnp.int32)


def kernel(x, gate_w, gate_b, expert_w, expert_b):
    pos2, prob2, counts2 = _gate(x, gate_w, gate_b)
    pos = pos2[:, 0]
    prob = prob2[:, 0]
    counts = counts2[0]

    ends = jnp.cumsum(counts)
    starts = ends - counts

    xs, ps = _dispatch(x, pos, prob)
    meta = _make_meta(counts, starts, ends)
    y = _grouped_mm(xs, ps, expert_w, expert_b, meta)
    return _combine(y, pos)
